# baseline (speedup 1.0000x reference)
import functools

import jax
import jax.numpy as jnp
import numpy as np
from jax import lax
from jax.experimental import pallas as pl
from jax.experimental.pallas import tpu as pltpu
from jax.experimental.pallas import tpu_sc as plsc

_VOCAB = 100000
_D = 1024
_B = 4
_S = 4096
_SCALE = np.float32(1.0 / np.sqrt(_D))

_NC = 2
_NS = 16
_NW = _NC * _NS
_L = 16

_N = _B * _S
_SPW = _S // _NW
_SR = 16
_NJ = _SPW // _SR
_NU = _NJ * _B
_PW = _NU * _SR


def _pos_encoding() -> np.ndarray:
    pos = np.arange(_S, dtype=np.float32)[:, None]
    div = np.exp(
        np.arange(0, _D, 2, dtype=np.float32) * (-np.log(10000.0) / _D)
    )
    pe = np.zeros((_S, _D), dtype=np.float32)
    pe[:, 0::2] = np.sin(pos * div)
    pe[:, 1::2] = np.cos(pos * div)
    return pe


_PE = _pos_encoding()


def _sc_body(idx_hbm, pe_hbm, table_hbm, out_hbm,
             idx_v, x0, x1, o0, o1, pe0, pe1,
             g0, g1, sc0, sc1, ps0, ps1):
    cid = lax.axis_index("c")
    sid = lax.axis_index("s")
    wid = sid * _NC + cid
    s0 = wid * _SPW

    xb = (x0, x1)
    ob = (o0, o1)
    peb = (pe0, pe1)
    gs = (g0, g1)
    ss = (sc0, sc1)
    ps = (ps0, ps1)

    pltpu.sync_copy(idx_hbm.at[pl.ds(wid * _PW, _PW)], idx_v)

    def fire_gather(u, k):
        pltpu.async_copy(
            table_hbm.at[idx_v.at[pl.ds(u * _SR, _SR)]], xb[k], gs[k])

    def wait_gather(u, k):
        pltpu.make_async_copy(
            table_hbm.at[idx_v.at[pl.ds(u * _SR, _SR)]], xb[k],
            gs[k]).wait()

    def fire_pe(j, kp):
        pltpu.async_copy(pe_hbm.at[pl.ds(s0 + j * _SR, _SR)], peb[kp],
                         ps[kp])

    def wait_pe(j, kp):
        pltpu.make_async_copy(pe_hbm.at[pl.ds(s0 + j * _SR, _SR)],
                              peb[kp], ps[kp]).wait()

    def out_slice(u):
        j = u // _B
        b = u % _B
        return out_hbm.at[pl.ds(b * _S + s0 + j * _SR, _SR)]

    def fire_scatter(u, k):
        pltpu.async_copy(ob[k], out_slice(u), ss[k])

    def wait_scatter(u, k):
        pltpu.make_async_copy(ob[k], out_slice(u), ss[k]).wait()

    def fma(k, kp):
        def row_body(r, carry):
            for col in range(_D // _L):
                sl = pl.ds(col * _L, _L)
                ob[k][r, sl] = xb[k][r, sl] * _SCALE + peb[kp][r, sl]
            return carry
        lax.fori_loop(0, _SR, row_body, 0)

    fire_pe(0, 0)
    fire_pe(1, 1)
    fire_gather(0, 0)
    fire_gather(1, 1)

    def iter_body(t, carry):
        for pos in range(8):
            k = pos % 2
            j_rel = pos // _B
            b = pos % _B
            u = 8 * t + pos
            j = 2 * t + j_rel

            wait_gather(u, k)
            if b == 0:
                wait_pe(j, j_rel)
            if pos >= 2:
                wait_scatter(u - 2, k)
            else:
                @pl.when(t > 0)
                def _():
                    wait_scatter(u - 2, k)
            fma(k, j_rel)
            if pos == 3:
                @pl.when(t < _NJ // 2 - 1)
                def _():
                    fire_pe(2 * t + 2, 0)
            if pos == 7:
                @pl.when(t < _NJ // 2 - 1)
                def _():
                    fire_pe(2 * t + 3, 1)
            if pos < 6:
                fire_gather(u + 2, k)
            else:
                @pl.when(t < _NJ // 2 - 1)
                def _():
                    fire_gather(u + 2, k)
            fire_scatter(u, k)
        return carry

    lax.fori_loop(0, _NJ // 2, iter_body, 0)

    wait_scatter(_NU - 2, 0)
    wait_scatter(_NU - 1, 1)


@jax.jit
def _embed(idx_r, table, pe):
    fn = functools.partial(
        pl.kernel,
        mesh=plsc.VectorSubcoreMesh(core_axis_name="c", subcore_axis_name="s"),
        out_type=jax.ShapeDtypeStruct((_N, _D), jnp.float32),
        scratch_types=[
            pltpu.VMEM((_PW,), jnp.int32),
            pltpu.VMEM((_SR, _D), jnp.float32),
            pltpu.VMEM((_SR, _D), jnp.float32),
            pltpu.VMEM((_SR, _D), jnp.float32),
            pltpu.VMEM((_SR, _D), jnp.float32),
            pltpu.VMEM((_SR, _D), jnp.float32),
            pltpu.VMEM((_SR, _D), jnp.float32),
            pltpu.SemaphoreType.DMA,
            pltpu.SemaphoreType.DMA,
            pltpu.SemaphoreType.DMA,
            pltpu.SemaphoreType.DMA,
            pltpu.SemaphoreType.DMA,
            pltpu.SemaphoreType.DMA,
        ],
    )(_sc_body)
    return fn(idx_r, pe, table)


def kernel(inputs, table):
    idx_r = (inputs.reshape(_B, _NW, _NJ, _SR)
             .transpose(1, 2, 0, 3)
             .reshape(_N))
    pe = jnp.asarray(_PE)
    out = _embed(idx_r, table, pe)
    return out.reshape(_B, _S, _D)


# speedup vs baseline: 1.1161x; 1.1161x over previous
import functools

import jax
import jax.numpy as jnp
import numpy as np
from jax import lax
from jax.experimental import pallas as pl
from jax.experimental.pallas import tpu as pltpu
from jax.experimental.pallas import tpu_sc as plsc

_VOCAB = 100000
_D = 1024
_B = 4
_S = 4096
_SCALE = np.float32(1.0 / np.sqrt(_D))

_NC = 2
_NS = 16
_NW = _NC * _NS
_L = 16

_N = _B * _S
_SPW = _S // _NW
_R = 8
_NCH = _SPW // _R
_CR = _B * _R
_PW = _NCH * _CR
_NBANK = 3


def _pos_encoding() -> np.ndarray:
    pos = np.arange(_S, dtype=np.float32)[:, None]
    div = np.exp(
        np.arange(0, _D, 2, dtype=np.float32) * (-np.log(10000.0) / _D)
    )
    pe = np.zeros((_S, _D), dtype=np.float32)
    pe[:, 0::2] = np.sin(pos * div)
    pe[:, 1::2] = np.cos(pos * div)
    return pe


_PE = _pos_encoding()


def _sc_body(idx_hbm, pe_hbm, table_hbm, out_hbm,
             idx_v, x0, x1, x2, g0, g1, g2, s0_, s1_, s2_):
    cid = lax.axis_index("c")
    sid = lax.axis_index("s")
    wid = sid * _NC + cid
    s0 = wid * _SPW

    xb = (x0, x1, x2)
    gs = (g0, g1, g2)
    ss = (s0_, s1_, s2_)

    pltpu.sync_copy(idx_hbm.at[pl.ds(wid * _PW, _PW)], idx_v)

    def fire_gather(c, k):
        pltpu.async_copy(pe_hbm.at[pl.ds(s0 + c * _R, _R)],
                         xb[k].at[pl.ds(_CR, _R)], gs[k])
        pltpu.async_copy(table_hbm.at[idx_v.at[pl.ds(c * _CR, _CR)]],
                         xb[k].at[pl.ds(0, _CR)], gs[k])

    def wait_gather(c, k):
        pltpu.make_async_copy(pe_hbm.at[pl.ds(s0 + c * _R, _R)],
                              xb[k].at[pl.ds(_CR, _R)], gs[k]).wait()
        pltpu.make_async_copy(table_hbm.at[idx_v.at[pl.ds(c * _CR, _CR)]],
                              xb[k].at[pl.ds(0, _CR)], gs[k]).wait()

    def fire_scatter(c, k):
        for b in range(_B):
            pltpu.async_copy(
                xb[k].at[pl.ds(b * _R, _R)],
                out_hbm.at[pl.ds(b * _S + s0 + c * _R, _R)], ss[k])

    def wait_scatter(c, k):
        for b in range(_B):
            pltpu.make_async_copy(
                xb[k].at[pl.ds(b * _R, _R)],
                out_hbm.at[pl.ds(b * _S + s0 + c * _R, _R)], ss[k]).wait()

    def fma(k):
        xk = xb[k]

        def row_body(r, carry):
            for col in range(_D // _L):
                sl = pl.ds(col * _L, _L)
                p = xk[_CR + r, sl]
                for b in range(_B):
                    row = b * _R + r
                    xk[row, sl] = xk[row, sl] * _SCALE + p
            return carry

        lax.fori_loop(0, _R, row_body, 0)

    def step(c, k, first, last_fire):
        wait_gather(c, k)
        fma(k)
        fire_scatter(c, k)
        kn = (k + 2) % _NBANK
        if first is None:
            wait_scatter(c - 1, kn)
        else:
            @pl.when(first > 0)
            def _():
                wait_scatter(c - 1, kn)
        if last_fire is None:
            fire_gather(c + 2, kn)
        else:
            @pl.when(last_fire)
            def _():
                fire_gather(c + 2, kn)

    fire_gather(0, 0)
    fire_gather(1, 1)

    def iter_body(t, carry):
        step(3 * t, 0, t, None)
        step(3 * t + 1, 1, None, None)
        step(3 * t + 2, 2, None, t < (_NCH // _NBANK - 1))
        return carry

    lax.fori_loop(0, _NCH // _NBANK, iter_body, 0)

    c = _NCH - 1
    wait_gather(c, 0)
    fma(0)
    fire_scatter(c, 0)
    wait_scatter(c - 1, 2)
    wait_scatter(c, 0)


@jax.jit
def _embed(idx_r, table, pe):
    fn = functools.partial(
        pl.kernel,
        mesh=plsc.VectorSubcoreMesh(core_axis_name="c", subcore_axis_name="s"),
        out_type=jax.ShapeDtypeStruct((_N, _D), jnp.float32),
        scratch_types=[
            pltpu.VMEM((_PW,), jnp.int32),
            pltpu.VMEM((_CR + _R, _D), jnp.float32),
            pltpu.VMEM((_CR + _R, _D), jnp.float32),
            pltpu.VMEM((_CR + _R, _D), jnp.float32),
            pltpu.SemaphoreType.DMA,
            pltpu.SemaphoreType.DMA,
            pltpu.SemaphoreType.DMA,
            pltpu.SemaphoreType.DMA,
            pltpu.SemaphoreType.DMA,
            pltpu.SemaphoreType.DMA,
        ],
    )(_sc_body)
    return fn(idx_r, pe, table)


def kernel(inputs, table):
    idx_r = (inputs.reshape(_B, _NW, _NCH, _R)
             .transpose(1, 2, 0, 3)
             .reshape(_N))
    pe = jnp.asarray(_PE)
    out = _embed(idx_r, table, pe)
    return out.reshape(_B, _S, _D)


# speedup vs baseline: 1.1307x; 1.0131x over previous
import functools

import jax
import jax.numpy as jnp
import numpy as np
from jax import lax
from jax.experimental import pallas as pl
from jax.experimental.pallas import tpu as pltpu
from jax.experimental.pallas import tpu_sc as plsc

_VOCAB = 100000
_D = 1024
_B = 4
_S = 4096
_SCALE = np.float32(1.0 / np.sqrt(_D))

_NC = 2
_NS = 16
_NW = _NC * _NS
_L = 16

_N = _B * _S
_SPW = _S // _NW
_R = 8
_NCH = _SPW // _R
_CR = _B * _R
_PW = _NCH * _CR
_NBANK = 3


def _pos_encoding() -> np.ndarray:
    pos = np.arange(_S, dtype=np.float32)[:, None]
    div = np.exp(
        np.arange(0, _D, 2, dtype=np.float32) * (-np.log(10000.0) / _D)
    )
    pe = np.zeros((_S, _D), dtype=np.float32)
    pe[:, 0::2] = np.sin(pos * div)
    pe[:, 1::2] = np.cos(pos * div)
    return pe


_PE = _pos_encoding()


def _sc_body(idx_hbm, pe_hbm, table_hbm, out_hbm,
             idx_v, x0, x1, x2, g0, g1, g2, s0_, s1_, s2_):
    cid = lax.axis_index("c")
    sid = lax.axis_index("s")
    wid = sid * _NC + cid
    s0 = wid * _SPW

    xb = (x0, x1, x2)
    gs = (g0, g1, g2)
    ss = (s0_, s1_, s2_)

    for b in range(_B):
        pltpu.sync_copy(idx_hbm.at[pl.ds(b * _S + s0, _SPW)],
                        idx_v.at[pl.ds(b * _SPW, _SPW)])

    def fire_gather(c, k):
        pltpu.async_copy(pe_hbm.at[pl.ds(s0 + c * _R, _R)],
                         xb[k].at[pl.ds(_CR, _R)], gs[k])
        for b in range(_B):
            pltpu.async_copy(
                table_hbm.at[idx_v.at[pl.ds(b * _SPW + c * _R, _R)]],
                xb[k].at[pl.ds(b * _R, _R)], gs[k])

    def wait_gather(c, k):
        pltpu.make_async_copy(pe_hbm.at[pl.ds(s0 + c * _R, _R)],
                              xb[k].at[pl.ds(_CR, _R)], gs[k]).wait()
        for b in range(_B):
            pltpu.make_async_copy(
                table_hbm.at[idx_v.at[pl.ds(b * _SPW + c * _R, _R)]],
                xb[k].at[pl.ds(b * _R, _R)], gs[k]).wait()

    def fire_scatter(c, k):
        for b in range(_B):
            pltpu.async_copy(
                xb[k].at[pl.ds(b * _R, _R)],
                out_hbm.at[pl.ds(b * _S + s0 + c * _R, _R)], ss[k])

    def wait_scatter(c, k):
        for b in range(_B):
            pltpu.make_async_copy(
                xb[k].at[pl.ds(b * _R, _R)],
                out_hbm.at[pl.ds(b * _S + s0 + c * _R, _R)], ss[k]).wait()

    def fma(k):
        xk = xb[k]

        def row_body(r, carry):
            for col in range(_D // _L):
                sl = pl.ds(col * _L, _L)
                p = xk[_CR + r, sl]
                for b in range(_B):
                    row = b * _R + r
                    xk[row, sl] = xk[row, sl] * _SCALE + p
            return carry

        lax.fori_loop(0, _R, row_body, 0)

    def step(c, k, first, last_fire):
        wait_gather(c, k)
        fma(k)
        fire_scatter(c, k)
        kn = (k + 2) % _NBANK
        if first is None:
            wait_scatter(c - 1, kn)
        else:
            @pl.when(first > 0)
            def _():
                wait_scatter(c - 1, kn)
        if last_fire is None:
            fire_gather(c + 2, kn)
        else:
            @pl.when(last_fire)
            def _():
                fire_gather(c + 2, kn)

    fire_gather(0, 0)
    fire_gather(1, 1)

    def iter_body(t, carry):
        step(3 * t, 0, t, None)
        step(3 * t + 1, 1, None, None)
        step(3 * t + 2, 2, None, t < (_NCH // _NBANK - 1))
        return carry

    lax.fori_loop(0, _NCH // _NBANK, iter_body, 0)

    c = _NCH - 1
    wait_gather(c, 0)
    fma(0)
    fire_scatter(c, 0)
    wait_scatter(c - 1, 2)
    wait_scatter(c, 0)


@jax.jit
def _embed(idx_r, table, pe):
    fn = functools.partial(
        pl.kernel,
        mesh=plsc.VectorSubcoreMesh(core_axis_name="c", subcore_axis_name="s"),
        out_type=jax.ShapeDtypeStruct((_N, _D), jnp.float32),
        scratch_types=[
            pltpu.VMEM((_PW,), jnp.int32),
            pltpu.VMEM((_CR + _R, _D), jnp.float32),
            pltpu.VMEM((_CR + _R, _D), jnp.float32),
            pltpu.VMEM((_CR + _R, _D), jnp.float32),
            pltpu.SemaphoreType.DMA,
            pltpu.SemaphoreType.DMA,
            pltpu.SemaphoreType.DMA,
            pltpu.SemaphoreType.DMA,
            pltpu.SemaphoreType.DMA,
            pltpu.SemaphoreType.DMA,
        ],
    )(_sc_body)
    return fn(idx_r, pe, table)


def kernel(inputs, table):
    idx_flat = inputs.reshape(_N)
    pe = jnp.asarray(_PE)
    out = _embed(idx_flat, table, pe)
    return out.reshape(_B, _S, _D)


# speedup vs baseline: 1.1317x; 1.0009x over previous
import functools

import jax
import jax.numpy as jnp
import numpy as np
from jax import lax
from jax.experimental import pallas as pl
from jax.experimental.pallas import tpu as pltpu
from jax.experimental.pallas import tpu_sc as plsc

_VOCAB = 100000
_D = 1024
_B = 4
_S = 4096
_SCALE = np.float32(1.0 / np.sqrt(_D))

_NC = 2
_NS = 16
_NW = _NC * _NS
_L = 16

_N = _B * _S
_SPW = _S // _NW
_R = 8
_NCH = _SPW // _R
_PW = _B * _SPW
_NBANK = 3


def _pos_encoding() -> np.ndarray:
    pos = np.arange(_S, dtype=np.float32)[:, None]
    div = np.exp(
        np.arange(0, _D, 2, dtype=np.float32) * (-np.log(10000.0) / _D)
    )
    pe = np.zeros((_S, _D), dtype=np.float32)
    pe[:, 0::2] = np.sin(pos * div)
    pe[:, 1::2] = np.cos(pos * div)
    return pe


_PE = _pos_encoding()


def _sc_body(idx_hbm, pe_hbm, table_hbm, out_hbm,
             idx_v, x0, x1, x2, g0, g1, g2, s0_, s1_, s2_):
    cid = lax.axis_index("c")
    sid = lax.axis_index("s")
    wid = sid * _NC + cid
    s0 = wid * _SPW

    xb = (x0, x1, x2)
    gs = (g0, g1, g2)
    ss = (s0_, s1_, s2_)

    for b in range(_B):
        pltpu.sync_copy(idx_hbm.at[pl.ds(b * _S + s0, _SPW)],
                        idx_v.at[pl.ds(b * _SPW, _SPW)])

    def fire_gather(c, k):
        pltpu.async_copy(pe_hbm.at[pl.ds(s0 + c * _R, _R)],
                         xb[k].at[_B], gs[k])
        for b in range(_B):
            pltpu.async_copy(
                table_hbm.at[idx_v.at[pl.ds(b * _SPW + c * _R, _R)]],
                xb[k].at[b], gs[k])

    def wait_gather(c, k):
        pltpu.make_async_copy(pe_hbm.at[pl.ds(s0 + c * _R, _R)],
                              xb[k].at[_B], gs[k]).wait()
        for b in range(_B):
            pltpu.make_async_copy(
                table_hbm.at[idx_v.at[pl.ds(b * _SPW + c * _R, _R)]],
                xb[k].at[b], gs[k]).wait()

    def fire_scatter(c, k):
        pltpu.async_copy(xb[k].at[pl.ds(0, _B)],
                         out_hbm.at[:, pl.ds(s0 + c * _R, _R), :], ss[k])

    def wait_scatter(c, k):
        pltpu.make_async_copy(xb[k].at[pl.ds(0, _B)],
                              out_hbm.at[:, pl.ds(s0 + c * _R, _R), :],
                              ss[k]).wait()

    def fma(k):
        xk = xb[k]

        def row_body(r, carry):
            for col in range(_D // _L):
                sl = pl.ds(col * _L, _L)
                p = xk[_B, r, sl]
                for b in range(_B):
                    xk[b, r, sl] = xk[b, r, sl] * _SCALE + p
            return carry

        lax.fori_loop(0, _R, row_body, 0)

    def step(c, k, first, last_fire):
        wait_gather(c, k)
        fma(k)
        fire_scatter(c, k)
        kn = (k + 2) % _NBANK
        if first is None:
            wait_scatter(c - 1, kn)
        else:
            @pl.when(first > 0)
            def _():
                wait_scatter(c - 1, kn)
        if last_fire is None:
            fire_gather(c + 2, kn)
        else:
            @pl.when(last_fire)
            def _():
                fire_gather(c + 2, kn)

    fire_gather(0, 0)
    fire_gather(1, 1)

    def iter_body(t, carry):
        step(3 * t, 0, t, None)
        step(3 * t + 1, 1, None, None)
        step(3 * t + 2, 2, None, t < (_NCH // _NBANK - 1))
        return carry

    lax.fori_loop(0, _NCH // _NBANK, iter_body, 0)

    c = _NCH - 1
    wait_gather(c, 0)
    fma(0)
    fire_scatter(c, 0)
    wait_scatter(c - 1, 2)
    wait_scatter(c, 0)


@jax.jit
def _embed(idx_flat, table, pe):
    fn = functools.partial(
        pl.kernel,
        mesh=plsc.VectorSubcoreMesh(core_axis_name="c", subcore_axis_name="s"),
        out_type=jax.ShapeDtypeStruct((_B, _S, _D), jnp.float32),
        scratch_types=[
            pltpu.VMEM((_PW,), jnp.int32),
            pltpu.VMEM((_B + 1, _R, _D), jnp.float32),
            pltpu.VMEM((_B + 1, _R, _D), jnp.float32),
            pltpu.VMEM((_B + 1, _R, _D), jnp.float32),
            pltpu.SemaphoreType.DMA,
            pltpu.SemaphoreType.DMA,
            pltpu.SemaphoreType.DMA,
            pltpu.SemaphoreType.DMA,
            pltpu.SemaphoreType.DMA,
            pltpu.SemaphoreType.DMA,
        ],
    )(_sc_body)
    return fn(idx_flat, pe, table)


def kernel(inputs, table):
    idx_flat = inputs.reshape(_N)
    pe = jnp.asarray(_PE)
    return _embed(idx_flat, table, pe)


# speedup vs baseline: 1.1335x; 1.0016x over previous
import functools

import jax
import jax.numpy as jnp
import numpy as np
from jax import lax
from jax.experimental import pallas as pl
from jax.experimental.pallas import tpu as pltpu
from jax.experimental.pallas import tpu_sc as plsc

_VOCAB = 100000
_D = 1024
_B = 4
_S = 4096
_SCALE = np.float32(1.0 / np.sqrt(_D))

_NC = 2
_NS = 16
_NW = _NC * _NS
_L = 16

_N = _B * _S
_SPW = _S // _NW
_R = 8
_NCH = _SPW // _R
_PW = _B * _SPW
_NBANK = 3


def _pos_encoding() -> np.ndarray:
    pos = np.arange(_S, dtype=np.float32)[:, None]
    div = np.exp(
        np.arange(0, _D, 2, dtype=np.float32) * (-np.log(10000.0) / _D)
    )
    pe = np.zeros((_S, _D), dtype=np.float32)
    pe[:, 0::2] = np.sin(pos * div)
    pe[:, 1::2] = np.cos(pos * div)
    return pe


_PE = jax.numpy.asarray(_pos_encoding())


def _sc_body(idx_hbm, pe_hbm, table_hbm, out_hbm,
             idx_v, x0, x1, x2, g0, g1, g2, s0_, s1_, s2_):
    cid = lax.axis_index("c")
    sid = lax.axis_index("s")
    wid = sid * _NC + cid
    s0 = wid * _SPW

    xb = (x0, x1, x2)
    gs = (g0, g1, g2)
    ss = (s0_, s1_, s2_)

    for b in range(_B):
        pltpu.sync_copy(idx_hbm.at[pl.ds(b * _S + s0, _SPW)],
                        idx_v.at[pl.ds(b * _SPW, _SPW)])

    def fire_gather(c, k):
        pltpu.async_copy(pe_hbm.at[pl.ds(s0 + c * _R, _R)],
                         xb[k].at[_B], gs[k])
        for b in range(_B):
            pltpu.async_copy(
                table_hbm.at[idx_v.at[pl.ds(b * _SPW + c * _R, _R)]],
                xb[k].at[b], gs[k])

    def wait_gather(c, k):
        pltpu.make_async_copy(pe_hbm.at[pl.ds(s0 + c * _R, _R)],
                              xb[k].at[_B], gs[k]).wait()
        for b in range(_B):
            pltpu.make_async_copy(
                table_hbm.at[idx_v.at[pl.ds(b * _SPW + c * _R, _R)]],
                xb[k].at[b], gs[k]).wait()

    def fire_scatter(c, k):
        pltpu.async_copy(xb[k].at[pl.ds(0, _B)],
                         out_hbm.at[:, pl.ds(s0 + c * _R, _R), :], ss[k])

    def wait_scatter(c, k):
        pltpu.make_async_copy(xb[k].at[pl.ds(0, _B)],
                              out_hbm.at[:, pl.ds(s0 + c * _R, _R), :],
                              ss[k]).wait()

    def fma(k):
        xk = xb[k]

        def row_body(r, carry):
            for col in range(_D // _L):
                sl = pl.ds(col * _L, _L)
                p = xk[_B, r, sl]
                for b in range(_B):
                    xk[b, r, sl] = xk[b, r, sl] * _SCALE + p
            return carry

        lax.fori_loop(0, _R, row_body, 0)

    def step(c, k, first, last_fire):
        wait_gather(c, k)
        fma(k)
        fire_scatter(c, k)
        kn = (k + 2) % _NBANK
        if first is None:
            wait_scatter(c - 1, kn)
        else:
            @pl.when(first > 0)
            def _():
                wait_scatter(c - 1, kn)
        if last_fire is None:
            fire_gather(c + 2, kn)
        else:
            @pl.when(last_fire)
            def _():
                fire_gather(c + 2, kn)

    fire_gather(0, 0)
    fire_gather(1, 1)

    def iter_body(t, carry):
        step(3 * t, 0, t, None)
        step(3 * t + 1, 1, None, None)
        step(3 * t + 2, 2, None, t < (_NCH // _NBANK - 1))
        return carry

    lax.fori_loop(0, _NCH // _NBANK, iter_body, 0)

    c = _NCH - 1
    wait_gather(c, 0)
    fma(0)
    fire_scatter(c, 0)
    wait_scatter(c - 1, 2)
    wait_scatter(c, 0)


@jax.jit
def _embed(idx_flat, table, pe):
    fn = functools.partial(
        pl.kernel,
        mesh=plsc.VectorSubcoreMesh(core_axis_name="c", subcore_axis_name="s"),
        out_type=jax.ShapeDtypeStruct((_B, _S, _D), jnp.float32),
        scratch_types=[
            pltpu.VMEM((_PW,), jnp.int32),
            pltpu.VMEM((_B + 1, _R, _D), jnp.float32),
            pltpu.VMEM((_B + 1, _R, _D), jnp.float32),
            pltpu.VMEM((_B + 1, _R, _D), jnp.float32),
            pltpu.SemaphoreType.DMA,
            pltpu.SemaphoreType.DMA,
            pltpu.SemaphoreType.DMA,
            pltpu.SemaphoreType.DMA,
            pltpu.SemaphoreType.DMA,
            pltpu.SemaphoreType.DMA,
        ],
    )(_sc_body)
    return fn(idx_flat, pe, table)


def kernel(inputs, table):
    idx_flat = inputs.reshape(_N)
    return _embed(idx_flat, table, _PE)
